# baseline (speedup 1.0000x reference)
import jax
import jax.numpy as jnp
from jax import lax
from jax.experimental import pallas as pl
from jax.experimental.pallas import tpu as pltpu
from jax.experimental.pallas import tpu_sc as plsc

B = 4
Q = 4096
HS = 64
WS = 64
S = HS * WS
DM = 256
H = 8
P = 4
DH = DM // H

SBLK = 512
QBLK = 512
K = 4 * H * P

QT = B * Q
NW = 32
CQ = 8
RPW = QT // NW
NIT = RPW // CQ

HP = lax.Precision.HIGHEST
DP = lax.Precision.DEFAULT


def _value_proj_body(enc_ref, wv_ref, bv_ref, tab_ref):
    x = jnp.dot(enc_ref[0], wv_ref[...], preferred_element_type=jnp.float32,
                precision=DP)
    x = x + bv_ref[0]
    for h in range(H):
        tab_ref[0, h, :, 0:DH] = x[:, h * DH:(h + 1) * DH]


def _value_proj(enc, w_val, b_val):
    return pl.pallas_call(
        _value_proj_body,
        grid=(B, S // SBLK),
        in_specs=[
            pl.BlockSpec((1, SBLK, DM), lambda b, s: (b, s, 0)),
            pl.BlockSpec((DM, DM), lambda b, s: (0, 0)),
            pl.BlockSpec((1, DM), lambda b, s: (0, 0)),
        ],
        out_specs=pl.BlockSpec((1, H, SBLK, 128), lambda b, s: (b, 0, s, 0)),
        out_shape=jax.ShapeDtypeStruct((B, H, S, 128), jnp.float32),
    )(enc, w_val, b_val)


def _sampling_body(hid_ref, ref_ref, wb_ref, bb_ref, wa_ref, ba_ref,
                   idx_ref, wts_ref):
    bpid = pl.program_id(0)
    hid = hid_ref[0]

    refx = ref_ref[0, :, 0:1] * float(WS)
    refy = ref_ref[0, :, 1:2] * float(HS)
    px = (jnp.dot(hid, wb_ref[:, :H * P], preferred_element_type=jnp.float32,
                  precision=HP) + refx + bb_ref[0, :H * P])
    py = (jnp.dot(hid, wb_ref[:, H * P:], preferred_element_type=jnp.float32,
                  precision=HP) + refy + bb_ref[0, H * P:])

    logit = jnp.dot(hid, wa_ref[...], preferred_element_type=jnp.float32,
                    precision=HP)
    logit = logit + ba_ref[0]
    ex = jnp.exp(logit)
    ri = lax.broadcasted_iota(jnp.int32, (H * P, H * P), 0)
    ci = lax.broadcasted_iota(jnp.int32, (H * P, H * P), 1)
    grp = (ri // P == ci // P).astype(jnp.float32)
    attn = ex / jnp.dot(ex, grp, preferred_element_type=jnp.float32,
                        precision=HP)

    x0 = jnp.floor(px)
    y0 = jnp.floor(py)
    fx = px - x0
    fy = py - y0
    vx0 = ((x0 >= 0) & (x0 <= WS - 1)).astype(jnp.float32)
    vx1 = ((x0 >= -1) & (x0 <= WS - 2)).astype(jnp.float32)
    vy0 = ((y0 >= 0) & (y0 <= HS - 1)).astype(jnp.float32)
    vy1 = ((y0 >= -1) & (y0 <= HS - 2)).astype(jnp.float32)
    cx0 = jnp.clip(x0, 0, WS - 1)
    cx1 = jnp.clip(x0 + 1, 0, WS - 1)
    cy0 = jnp.clip(y0, 0, HS - 1)
    cy1 = jnp.clip(y0 + 1, 0, HS - 1)
    wx0 = (1.0 - fx) * vx0
    wx1 = fx * vx1
    wy0 = (1.0 - fy) * vy0
    wy1 = fy * vy1

    hlane = lax.broadcasted_iota(jnp.int32, (QBLK, H * P), 1) // P
    base = (bpid * H + hlane).astype(jnp.float32) * float(4 * S)

    wparts, iparts = [], []
    for wy, cy in ((wy0, cy0), (wy1, cy1)):
        for wx, cx in ((wx0, cx0), (wx1, cx1)):
            wparts.append(attn * wy * wx)
            iparts.append(base + cy * (4.0 * WS) + cx * 4.0)
    wts_ref[0] = jnp.concatenate(wparts, axis=1)
    idx_ref[0] = jnp.concatenate(iparts, axis=1).astype(jnp.int32)


def _sampling(hid, refp, w_big, b_big, w_attn, b_attn):
    return pl.pallas_call(
        _sampling_body,
        grid=(B, Q // QBLK),
        in_specs=[
            pl.BlockSpec((1, QBLK, DM), lambda b, q: (b, q, 0)),
            pl.BlockSpec((1, QBLK, 2), lambda b, q: (b, q, 0)),
            pl.BlockSpec((DM, 2 * H * P), lambda b, q: (0, 0)),
            pl.BlockSpec((1, 2 * H * P), lambda b, q: (0, 0)),
            pl.BlockSpec((DM, H * P), lambda b, q: (0, 0)),
            pl.BlockSpec((1, H * P), lambda b, q: (0, 0)),
        ],
        out_specs=[
            pl.BlockSpec((1, QBLK, K), lambda b, q: (b, q, 0)),
            pl.BlockSpec((1, QBLK, K), lambda b, q: (b, q, 0)),
        ],
        out_shape=[
            jax.ShapeDtypeStruct((B, Q, K), jnp.int32),
            jax.ShapeDtypeStruct((B, Q, K), jnp.float32),
        ],
    )(hid, refp, w_big, b_big, w_attn, b_attn)


def _sc_body(tab_hbm, idx_hbm, w_hbm, outa_hbm, outb_hbm,
             idx_v, w_v, w_safe, rows_v, out_v, sg0, sg1, si0, si1):
    wid = lax.axis_index("s") * 2 + lax.axis_index("c")
    base_q = wid * RPW
    sg = (sg0, sg1)
    si = (si0, si1)

    def fire_iw(g, par):
        q0 = base_q + g * CQ
        pltpu.async_copy(idx_hbm.at[pl.ds(q0, CQ)], idx_v.at[par], si[par])
        pltpu.async_copy(w_hbm.at[pl.ds(q0, CQ)], w_v.at[par], si[par])

    def wait_iw(g, par):
        q0 = base_q + g * CQ
        pltpu.make_async_copy(idx_hbm.at[pl.ds(q0, CQ)], idx_v.at[par],
                              si[par]).wait()
        pltpu.make_async_copy(w_hbm.at[pl.ds(q0, CQ)], w_v.at[par],
                              si[par]).wait()

    def fire_gathers(par):
        for c in range(CQ):
            pltpu.async_copy(tab_hbm.at[idx_v.at[par, c]],
                             rows_v.at[par, pl.ds(c * K, K)], sg[par])

    def wait_gathers(par):
        for c in range(CQ):
            pltpu.make_async_copy(tab_hbm.at[idx_v.at[par, c]],
                                  rows_v.at[par, pl.ds(c * K, K)],
                                  sg[par]).wait()

    def combine_write(g, par):
        q0 = base_q + g * CQ

        def one_q(c, _):
            wvec = [w_safe[c, pl.ds(j * 16, 16)] for j in range(K // 16)]
            for h in range(H):
                a0 = jnp.zeros((16,), jnp.float32)
                a1 = jnp.zeros((16,), jnp.float32)
                b0 = jnp.zeros((16,), jnp.float32)
                b1 = jnp.zeros((16,), jnp.float32)
                for corner in range(4):
                    for p in range(P):
                        lane = corner * (H * P) + h * P + p
                        ws = wvec[lane // 16][lane % 16]
                        r = c * K + lane
                        g0 = rows_v[par, r, pl.ds(0, 16)]
                        g1 = rows_v[par, r, pl.ds(16, 16)]
                        if (corner & 1) == 0:
                            a0 = a0 + ws * g0
                            a1 = a1 + ws * g1
                        else:
                            b0 = b0 + ws * g0
                            b1 = b1 + ws * g1
                half = h // 4
                off = (h % 4) * DH
                out_v[half, c, pl.ds(off, 16)] = a0 + b0
                out_v[half, c, pl.ds(off + 16, 16)] = a1 + b1
            return ()

        lax.fori_loop(0, CQ, one_q, ())
        pltpu.sync_copy(out_v.at[0], outa_hbm.at[pl.ds(q0, CQ)])
        pltpu.sync_copy(out_v.at[1], outb_hbm.at[pl.ds(q0, CQ)])

    fire_iw(0, 0)
    wait_iw(0, 0)
    fire_gathers(0)
    fire_iw(1, 1)

    def step(g, _):
        par = lax.rem(g, 2)

        @pl.when(par == 0)
        def _():
            _half(g, 0)

        @pl.when(par == 1)
        def _():
            _half(g, 1)

        return ()

    def _half(g, par):
        @pl.when(g + 1 < NIT)
        def _():
            wait_iw(g + 1, 1 - par)
            fire_gathers(1 - par)
        wait_gathers(par)
        def snap(c, _):
            for j in range(K // 16):
                w_safe[c, pl.ds(j * 16, 16)] = w_v[par, c, pl.ds(j * 16, 16)]
            return ()

        lax.fori_loop(0, CQ, snap, ())

        @pl.when(g + 2 < NIT)
        def _():
            fire_iw(g + 2, par)
        combine_write(g, par)

    lax.fori_loop(0, NIT, step, ())


def _sc_gather_combine(table, idx, wts):
    fn = pl.kernel(
        _sc_body,
        out_type=[
            jax.ShapeDtypeStruct((QT, 128), jnp.float32),
            jax.ShapeDtypeStruct((QT, 128), jnp.float32),
        ],
        mesh=plsc.VectorSubcoreMesh(core_axis_name="c", subcore_axis_name="s"),
        scratch_types=[
            pltpu.VMEM((2, CQ, K), jnp.int32),
            pltpu.VMEM((2, CQ, K), jnp.float32),
            pltpu.VMEM((CQ, K), jnp.float32),
            pltpu.VMEM((2, CQ * K, DH), jnp.float32),
            pltpu.VMEM((2, CQ, 128), jnp.float32),
            pltpu.SemaphoreType.DMA,
            pltpu.SemaphoreType.DMA,
            pltpu.SemaphoreType.DMA,
            pltpu.SemaphoreType.DMA,
        ],
        compiler_params=pltpu.CompilerParams(use_tc_tiling_on_sc=False),
    )
    return fn(table, idx, wts)


def _out_proj_body(xa_ref, xb_ref, wo_ref, bo_ref, o_ref):
    o_ref[...] = (
        jnp.dot(xa_ref[...], wo_ref[:128], preferred_element_type=jnp.float32,
                precision=DP)
        + jnp.dot(xb_ref[...], wo_ref[128:], preferred_element_type=jnp.float32,
                  precision=DP)
        + bo_ref[0]
    )


def _out_proj(xa, xb, w_out, b_out):
    return pl.pallas_call(
        _out_proj_body,
        grid=(QT // QBLK,),
        in_specs=[
            pl.BlockSpec((QBLK, 128), lambda i: (i, 0)),
            pl.BlockSpec((QBLK, 128), lambda i: (i, 0)),
            pl.BlockSpec((DM, DM), lambda i: (0, 0)),
            pl.BlockSpec((1, DM), lambda i: (0, 0)),
        ],
        out_specs=pl.BlockSpec((QBLK, DM), lambda i: (i, 0)),
        out_shape=jax.ShapeDtypeStruct((QT, DM), jnp.float32),
    )(xa, xb, w_out, b_out)


def kernel(hidden_states, encoder_hidden_states, reference_points,
           spatial_shapes, W_off, b_off, W_attn, b_attn, W_val, b_val,
           W_out, b_out):
    del spatial_shapes
    wo4 = W_off.reshape(DM, H, P, 2)
    w_big = jnp.concatenate(
        [wo4[..., 0].reshape(DM, H * P), wo4[..., 1].reshape(DM, H * P)],
        axis=1)
    bo4 = b_off.reshape(H, P, 2)
    b_big = jnp.concatenate(
        [bo4[..., 0].reshape(H * P), bo4[..., 1].reshape(H * P)]) - 0.5

    table = _value_proj(encoder_hidden_states, W_val, b_val.reshape(1, DM))
    idx, wts = _sampling(hidden_states, reference_points.reshape(B, Q, 2),
                         w_big, b_big.reshape(1, 2 * H * P),
                         W_attn, b_attn.reshape(1, H * P))
    attn_a, attn_b = _sc_gather_combine(table.reshape(B * H * S * 4, DH),
                                        idx.reshape(QT, K), wts.reshape(QT, K))
    out = _out_proj(attn_a, attn_b, W_out, b_out.reshape(1, DM))
    return out.reshape(B, Q, DM)


# speedup vs baseline: 1.1516x; 1.1516x over previous
import jax
import jax.numpy as jnp
from jax import lax
from jax.experimental import pallas as pl
from jax.experimental.pallas import tpu as pltpu
from jax.experimental.pallas import tpu_sc as plsc

B = 4
Q = 4096
HS = 64
WS = 64
S = HS * WS
DM = 256
H = 8
P = 4
DH = DM // H

SBLK = 512
QBLK = 512
K = 4 * H * P

QT = B * Q
QH = QT // 2
NW = 32
CQ = 8
RPW = QH // NW
NIT = RPW // CQ

HP = lax.Precision.HIGHEST
DP = lax.Precision.DEFAULT


def _value_proj_body(enc_ref, wv_ref, bv_ref, tab_ref):
    x = jnp.dot(enc_ref[0], wv_ref[...], preferred_element_type=jnp.float32,
                precision=DP)
    x = x + bv_ref[0]
    for h in range(H):
        tab_ref[0, h, :, 0:DH] = x[:, h * DH:(h + 1) * DH]


def _value_proj(enc, w_val, b_val, b0):
    return pl.pallas_call(
        _value_proj_body,
        grid=(B // 2, S // SBLK),
        in_specs=[
            pl.BlockSpec((1, SBLK, DM), lambda b, s: (b + b0, s, 0)),
            pl.BlockSpec((DM, DM), lambda b, s: (0, 0)),
            pl.BlockSpec((1, DM), lambda b, s: (0, 0)),
        ],
        out_specs=pl.BlockSpec((1, H, SBLK, 128), lambda b, s: (b, 0, s, 0)),
        out_shape=jax.ShapeDtypeStruct((B // 2, H, S, 128), jnp.float32),
    )(enc, w_val, b_val)


def _sampling_body(hid_ref, ref_ref, wb_ref, bb_ref, wa_ref, ba_ref,
                   idx_ref, wts_ref):
    bpid = pl.program_id(0)
    hid = hid_ref[0]

    er = lax.broadcasted_iota(jnp.int32, (2, 2 * H * P), 0)
    ec = lax.broadcasted_iota(jnp.int32, (2, 2 * H * P), 1)
    sel = ((er == 0) & (ec < H * P)) | ((er == 1) & (ec >= H * P))
    emat = jnp.where(sel, float(WS), 0.0)
    pxy = (jnp.dot(hid, wb_ref[...], preferred_element_type=jnp.float32,
                   precision=HP)
           + jnp.dot(ref_ref[0], emat, preferred_element_type=jnp.float32,
                     precision=HP)
           + bb_ref[0])
    px = pxy[:, :H * P]
    py = pxy[:, H * P:]

    logit = jnp.dot(hid, wa_ref[...], preferred_element_type=jnp.float32,
                    precision=HP)
    logit = logit + ba_ref[0]
    ex = jnp.exp(logit)
    ri = lax.broadcasted_iota(jnp.int32, (H * P, H * P), 0)
    ci = lax.broadcasted_iota(jnp.int32, (H * P, H * P), 1)
    grp = (ri // P == ci // P).astype(jnp.float32)
    attn = ex / jnp.dot(ex, grp, preferred_element_type=jnp.float32,
                        precision=HP)

    x0 = jnp.floor(px)
    y0 = jnp.floor(py)
    fx = px - x0
    fy = py - y0
    vx0 = ((x0 >= 0) & (x0 <= WS - 1)).astype(jnp.float32)
    vx1 = ((x0 >= -1) & (x0 <= WS - 2)).astype(jnp.float32)
    vy0 = ((y0 >= 0) & (y0 <= HS - 1)).astype(jnp.float32)
    vy1 = ((y0 >= -1) & (y0 <= HS - 2)).astype(jnp.float32)
    cx0 = jnp.clip(x0, 0, WS - 1)
    cx1 = jnp.clip(x0 + 1, 0, WS - 1)
    cy0 = jnp.clip(y0, 0, HS - 1)
    cy1 = jnp.clip(y0 + 1, 0, HS - 1)
    wx0 = (1.0 - fx) * vx0
    wx1 = fx * vx1
    wy0 = (1.0 - fy) * vy0
    wy1 = fy * vy1

    hlane = lax.broadcasted_iota(jnp.int32, (QBLK, H * P), 1) // P
    base = (bpid * H + hlane).astype(jnp.float32) * float(4 * S)

    wparts, iparts = [], []
    for wy, cy in ((wy0, cy0), (wy1, cy1)):
        for wx, cx in ((wx0, cx0), (wx1, cx1)):
            wparts.append(attn * wy * wx)
            iparts.append(base + cy * (4.0 * WS) + cx * 4.0)
    wts_ref[0] = jnp.concatenate(wparts, axis=1)
    idx_ref[0] = jnp.concatenate(iparts, axis=1).astype(jnp.int32)


def _sampling(hid, refp, w_big, b_big, w_attn, b_attn, b0):
    return pl.pallas_call(
        _sampling_body,
        grid=(B // 2, Q // QBLK),
        in_specs=[
            pl.BlockSpec((1, QBLK, DM), lambda b, q: (b + b0, q, 0)),
            pl.BlockSpec((1, QBLK, 2), lambda b, q: (b + b0, q, 0)),
            pl.BlockSpec((DM, 2 * H * P), lambda b, q: (0, 0)),
            pl.BlockSpec((1, 2 * H * P), lambda b, q: (0, 0)),
            pl.BlockSpec((DM, H * P), lambda b, q: (0, 0)),
            pl.BlockSpec((1, H * P), lambda b, q: (0, 0)),
        ],
        out_specs=[
            pl.BlockSpec((1, QBLK, K), lambda b, q: (b, q, 0)),
            pl.BlockSpec((1, QBLK, K), lambda b, q: (b, q, 0)),
        ],
        out_shape=[
            jax.ShapeDtypeStruct((B // 2, Q, K), jnp.int32),
            jax.ShapeDtypeStruct((B // 2, Q, K), jnp.float32),
        ],
    )(hid, refp, w_big, b_big, w_attn, b_attn)


def _sc_body(tab_hbm, idx_hbm, w_hbm, outa_hbm, outb_hbm,
             idx_v, w_v, w_safe, rows_v, out_v, sg0, sg1, si0, si1):
    wid = lax.axis_index("s") * 2 + lax.axis_index("c")
    base_q = wid * RPW
    sg = (sg0, sg1)
    si = (si0, si1)

    def fire_iw(g, par):
        q0 = base_q + g * CQ
        pltpu.async_copy(idx_hbm.at[pl.ds(q0, CQ)], idx_v.at[par], si[par])
        pltpu.async_copy(w_hbm.at[pl.ds(q0, CQ)], w_v.at[par], si[par])

    def wait_iw(g, par):
        q0 = base_q + g * CQ
        pltpu.make_async_copy(idx_hbm.at[pl.ds(q0, CQ)], idx_v.at[par],
                              si[par]).wait()
        pltpu.make_async_copy(w_hbm.at[pl.ds(q0, CQ)], w_v.at[par],
                              si[par]).wait()

    def fire_gathers(par):
        for c in range(CQ):
            pltpu.async_copy(tab_hbm.at[idx_v.at[par, c]],
                             rows_v.at[par, pl.ds(c * K, K)], sg[par])

    def wait_gathers(par):
        for c in range(CQ):
            pltpu.make_async_copy(tab_hbm.at[idx_v.at[par, c]],
                                  rows_v.at[par, pl.ds(c * K, K)],
                                  sg[par]).wait()

    def combine_write(g, par):
        q0 = base_q + g * CQ

        def one_q(c, _):
            wvec = [w_safe[c, pl.ds(j * 16, 16)] for j in range(K // 16)]
            for h in range(H):
                a0 = jnp.zeros((16,), jnp.float32)
                a1 = jnp.zeros((16,), jnp.float32)
                b0 = jnp.zeros((16,), jnp.float32)
                b1 = jnp.zeros((16,), jnp.float32)
                for corner in range(4):
                    for p in range(P):
                        lane = corner * (H * P) + h * P + p
                        ws = wvec[lane // 16][lane % 16]
                        r = c * K + lane
                        g0 = rows_v[par, r, pl.ds(0, 16)]
                        g1 = rows_v[par, r, pl.ds(16, 16)]
                        if (corner & 1) == 0:
                            a0 = a0 + ws * g0
                            a1 = a1 + ws * g1
                        else:
                            b0 = b0 + ws * g0
                            b1 = b1 + ws * g1
                half = h // 4
                off = (h % 4) * DH
                out_v[half, c, pl.ds(off, 16)] = a0 + b0
                out_v[half, c, pl.ds(off + 16, 16)] = a1 + b1
            return ()

        lax.fori_loop(0, CQ, one_q, ())
        pltpu.sync_copy(out_v.at[0], outa_hbm.at[pl.ds(q0, CQ)])
        pltpu.sync_copy(out_v.at[1], outb_hbm.at[pl.ds(q0, CQ)])

    fire_iw(0, 0)
    wait_iw(0, 0)
    fire_gathers(0)
    fire_iw(1, 1)

    def step(g, _):
        par = lax.rem(g, 2)

        @pl.when(par == 0)
        def _():
            _half(g, 0)

        @pl.when(par == 1)
        def _():
            _half(g, 1)

        return ()

    def _half(g, par):
        @pl.when(g + 1 < NIT)
        def _():
            wait_iw(g + 1, 1 - par)
            fire_gathers(1 - par)
        wait_gathers(par)
        def snap(c, _):
            for j in range(K // 16):
                w_safe[c, pl.ds(j * 16, 16)] = w_v[par, c, pl.ds(j * 16, 16)]
            return ()

        lax.fori_loop(0, CQ, snap, ())

        @pl.when(g + 2 < NIT)
        def _():
            fire_iw(g + 2, par)
        combine_write(g, par)

    lax.fori_loop(0, NIT, step, ())


def _sc_gather_combine(table, idx, wts):
    fn = pl.kernel(
        _sc_body,
        out_type=[
            jax.ShapeDtypeStruct((QH, 128), jnp.float32),
            jax.ShapeDtypeStruct((QH, 128), jnp.float32),
        ],
        mesh=plsc.VectorSubcoreMesh(core_axis_name="c", subcore_axis_name="s"),
        scratch_types=[
            pltpu.VMEM((2, CQ, K), jnp.int32),
            pltpu.VMEM((2, CQ, K), jnp.float32),
            pltpu.VMEM((CQ, K), jnp.float32),
            pltpu.VMEM((2, CQ * K, DH), jnp.float32),
            pltpu.VMEM((2, CQ, 128), jnp.float32),
            pltpu.SemaphoreType.DMA,
            pltpu.SemaphoreType.DMA,
            pltpu.SemaphoreType.DMA,
            pltpu.SemaphoreType.DMA,
        ],
        compiler_params=pltpu.CompilerParams(use_tc_tiling_on_sc=False),
    )
    return fn(table, idx, wts)


def _out_proj_body(xa_ref, xb_ref, wo_ref, bo_ref, o_ref):
    o_ref[...] = (
        jnp.dot(xa_ref[...], wo_ref[:128], preferred_element_type=jnp.float32,
                precision=DP)
        + jnp.dot(xb_ref[...], wo_ref[128:], preferred_element_type=jnp.float32,
                  precision=DP)
        + bo_ref[0]
    )


def _out_proj(xa, xb, w_out, b_out):
    return pl.pallas_call(
        _out_proj_body,
        grid=(QH // QBLK,),
        in_specs=[
            pl.BlockSpec((QBLK, 128), lambda i: (i, 0)),
            pl.BlockSpec((QBLK, 128), lambda i: (i, 0)),
            pl.BlockSpec((DM, DM), lambda i: (0, 0)),
            pl.BlockSpec((1, DM), lambda i: (0, 0)),
        ],
        out_specs=pl.BlockSpec((QBLK, DM), lambda i: (i, 0)),
        out_shape=jax.ShapeDtypeStruct((QH, DM), jnp.float32),
    )(xa, xb, w_out, b_out)


def kernel(hidden_states, encoder_hidden_states, reference_points,
           spatial_shapes, W_off, b_off, W_attn, b_attn, W_val, b_val,
           W_out, b_out):
    del spatial_shapes
    wo4 = W_off.reshape(DM, H, P, 2)
    w_big = jnp.concatenate(
        [wo4[..., 0].reshape(DM, H * P), wo4[..., 1].reshape(DM, H * P)],
        axis=1)
    bo4 = b_off.reshape(H, P, 2)
    b_big = jnp.concatenate(
        [bo4[..., 0].reshape(H * P), bo4[..., 1].reshape(H * P)]) - 0.5

    halves = []
    for b0 in (0, B // 2):
        table = _value_proj(encoder_hidden_states, W_val,
                            b_val.reshape(1, DM), b0)
        idx, wts = _sampling(hidden_states,
                             reference_points.reshape(B, Q, 2),
                             w_big, b_big.reshape(1, 2 * H * P),
                             W_attn, b_attn.reshape(1, H * P), b0)
        attn_a, attn_b = _sc_gather_combine(
            table.reshape((B // 2) * H * S * 4, DH),
            idx.reshape(QH, K), wts.reshape(QH, K))
        halves.append(_out_proj(attn_a, attn_b, W_out, b_out.reshape(1, DM)))
    return jnp.concatenate(halves, axis=0).reshape(B, Q, DM)


# speedup vs baseline: 1.1872x; 1.0309x over previous
import jax
import jax.numpy as jnp
from jax import lax
from jax.experimental import pallas as pl
from jax.experimental.pallas import tpu as pltpu
from jax.experimental.pallas import tpu_sc as plsc

B = 4
Q = 4096
HS = 64
WS = 64
S = HS * WS
DM = 256
H = 8
P = 4
DH = DM // H

SBLK = 512
QBLK = 512
K = 4 * H * P

QT = B * Q
QH = QT // 4
NW = 32
CQ = 8
RPW = QH // NW
NIT = RPW // CQ

HP = lax.Precision.HIGHEST
DP = lax.Precision.DEFAULT


def _value_proj_body(enc_ref, wv_ref, bv_ref, tab_ref):
    x = jnp.dot(enc_ref[0], wv_ref[...], preferred_element_type=jnp.float32,
                precision=DP)
    x = x + bv_ref[0]
    for h in range(H):
        tab_ref[0, h, :, 0:DH] = x[:, h * DH:(h + 1) * DH]


def _value_proj(enc, w_val, b_val, b0):
    return pl.pallas_call(
        _value_proj_body,
        grid=(B // 4, S // SBLK),
        in_specs=[
            pl.BlockSpec((1, SBLK, DM), lambda b, s: (b + b0, s, 0)),
            pl.BlockSpec((DM, DM), lambda b, s: (0, 0)),
            pl.BlockSpec((1, DM), lambda b, s: (0, 0)),
        ],
        out_specs=pl.BlockSpec((1, H, SBLK, 128), lambda b, s: (b, 0, s, 0)),
        out_shape=jax.ShapeDtypeStruct((B // 4, H, S, 128), jnp.float32),
    )(enc, w_val, b_val)


def _sampling_body(hid_ref, ref_ref, wb_ref, bb_ref, wa_ref, ba_ref,
                   idx_ref, wts_ref):
    bpid = pl.program_id(0)
    hid = hid_ref[0]

    er = lax.broadcasted_iota(jnp.int32, (2, 2 * H * P), 0)
    ec = lax.broadcasted_iota(jnp.int32, (2, 2 * H * P), 1)
    sel = ((er == 0) & (ec < H * P)) | ((er == 1) & (ec >= H * P))
    emat = jnp.where(sel, float(WS), 0.0)
    pxy = (jnp.dot(hid, wb_ref[...], preferred_element_type=jnp.float32,
                   precision=HP)
           + jnp.dot(ref_ref[0], emat, preferred_element_type=jnp.float32,
                     precision=HP)
           + bb_ref[0])
    px = pxy[:, :H * P]
    py = pxy[:, H * P:]

    logit = jnp.dot(hid, wa_ref[...], preferred_element_type=jnp.float32,
                    precision=HP)
    logit = logit + ba_ref[0]
    ex = jnp.exp(logit)
    ri = lax.broadcasted_iota(jnp.int32, (H * P, H * P), 0)
    ci = lax.broadcasted_iota(jnp.int32, (H * P, H * P), 1)
    grp = (ri // P == ci // P).astype(jnp.float32)
    attn = ex / jnp.dot(ex, grp, preferred_element_type=jnp.float32,
                        precision=HP)

    x0 = jnp.floor(px)
    y0 = jnp.floor(py)
    fx = px - x0
    fy = py - y0
    vx0 = ((x0 >= 0) & (x0 <= WS - 1)).astype(jnp.float32)
    vx1 = ((x0 >= -1) & (x0 <= WS - 2)).astype(jnp.float32)
    vy0 = ((y0 >= 0) & (y0 <= HS - 1)).astype(jnp.float32)
    vy1 = ((y0 >= -1) & (y0 <= HS - 2)).astype(jnp.float32)
    cx0 = jnp.clip(x0, 0, WS - 1)
    cx1 = jnp.clip(x0 + 1, 0, WS - 1)
    cy0 = jnp.clip(y0, 0, HS - 1)
    cy1 = jnp.clip(y0 + 1, 0, HS - 1)
    wx0 = (1.0 - fx) * vx0
    wx1 = fx * vx1
    wy0 = (1.0 - fy) * vy0
    wy1 = fy * vy1

    hlane = lax.broadcasted_iota(jnp.int32, (QBLK, H * P), 1) // P
    base = (bpid * H + hlane).astype(jnp.float32) * float(4 * S)

    wparts, iparts = [], []
    for wy, cy in ((wy0, cy0), (wy1, cy1)):
        for wx, cx in ((wx0, cx0), (wx1, cx1)):
            wparts.append(attn * wy * wx)
            iparts.append(base + cy * (4.0 * WS) + cx * 4.0)
    wts_ref[0] = jnp.concatenate(wparts, axis=1)
    idx_ref[0] = jnp.concatenate(iparts, axis=1).astype(jnp.int32)


def _sampling(hid, refp, w_big, b_big, w_attn, b_attn, b0):
    return pl.pallas_call(
        _sampling_body,
        grid=(B // 4, Q // QBLK),
        in_specs=[
            pl.BlockSpec((1, QBLK, DM), lambda b, q: (b + b0, q, 0)),
            pl.BlockSpec((1, QBLK, 2), lambda b, q: (b + b0, q, 0)),
            pl.BlockSpec((DM, 2 * H * P), lambda b, q: (0, 0)),
            pl.BlockSpec((1, 2 * H * P), lambda b, q: (0, 0)),
            pl.BlockSpec((DM, H * P), lambda b, q: (0, 0)),
            pl.BlockSpec((1, H * P), lambda b, q: (0, 0)),
        ],
        out_specs=[
            pl.BlockSpec((1, QBLK, K), lambda b, q: (b, q, 0)),
            pl.BlockSpec((1, QBLK, K), lambda b, q: (b, q, 0)),
        ],
        out_shape=[
            jax.ShapeDtypeStruct((B // 4, Q, K), jnp.int32),
            jax.ShapeDtypeStruct((B // 4, Q, K), jnp.float32),
        ],
    )(hid, refp, w_big, b_big, w_attn, b_attn)


def _sc_body(tab_hbm, idx_hbm, w_hbm, outa_hbm, outb_hbm,
             idx_v, w_v, w_safe, rows_v, out_v, sg0, sg1, si0, si1):
    wid = lax.axis_index("s") * 2 + lax.axis_index("c")
    base_q = wid * RPW
    sg = (sg0, sg1)
    si = (si0, si1)

    def fire_iw(g, par):
        q0 = base_q + g * CQ
        pltpu.async_copy(idx_hbm.at[pl.ds(q0, CQ)], idx_v.at[par], si[par])
        pltpu.async_copy(w_hbm.at[pl.ds(q0, CQ)], w_v.at[par], si[par])

    def wait_iw(g, par):
        q0 = base_q + g * CQ
        pltpu.make_async_copy(idx_hbm.at[pl.ds(q0, CQ)], idx_v.at[par],
                              si[par]).wait()
        pltpu.make_async_copy(w_hbm.at[pl.ds(q0, CQ)], w_v.at[par],
                              si[par]).wait()

    def fire_gathers(par):
        for c in range(CQ):
            pltpu.async_copy(tab_hbm.at[idx_v.at[par, c]],
                             rows_v.at[par, pl.ds(c * K, K)], sg[par])

    def wait_gathers(par):
        for c in range(CQ):
            pltpu.make_async_copy(tab_hbm.at[idx_v.at[par, c]],
                                  rows_v.at[par, pl.ds(c * K, K)],
                                  sg[par]).wait()

    def combine_write(g, par):
        q0 = base_q + g * CQ

        def one_q(c, _):
            wvec = [w_safe[c, pl.ds(j * 16, 16)] for j in range(K // 16)]
            for h in range(H):
                a0 = jnp.zeros((16,), jnp.float32)
                a1 = jnp.zeros((16,), jnp.float32)
                b0 = jnp.zeros((16,), jnp.float32)
                b1 = jnp.zeros((16,), jnp.float32)
                for corner in range(4):
                    for p in range(P):
                        lane = corner * (H * P) + h * P + p
                        ws = wvec[lane // 16][lane % 16]
                        r = c * K + lane
                        g0 = rows_v[par, r, pl.ds(0, 16)]
                        g1 = rows_v[par, r, pl.ds(16, 16)]
                        if (corner & 1) == 0:
                            a0 = a0 + ws * g0
                            a1 = a1 + ws * g1
                        else:
                            b0 = b0 + ws * g0
                            b1 = b1 + ws * g1
                half = h // 4
                off = (h % 4) * DH
                out_v[half, c, pl.ds(off, 16)] = a0 + b0
                out_v[half, c, pl.ds(off + 16, 16)] = a1 + b1
            return ()

        lax.fori_loop(0, CQ, one_q, ())
        pltpu.sync_copy(out_v.at[0], outa_hbm.at[pl.ds(q0, CQ)])
        pltpu.sync_copy(out_v.at[1], outb_hbm.at[pl.ds(q0, CQ)])

    fire_iw(0, 0)
    wait_iw(0, 0)
    fire_gathers(0)
    fire_iw(1, 1)

    def step(g, _):
        par = lax.rem(g, 2)

        @pl.when(par == 0)
        def _():
            _half(g, 0)

        @pl.when(par == 1)
        def _():
            _half(g, 1)

        return ()

    def _half(g, par):
        @pl.when(g + 1 < NIT)
        def _():
            wait_iw(g + 1, 1 - par)
            fire_gathers(1 - par)
        wait_gathers(par)
        def snap(c, _):
            for j in range(K // 16):
                w_safe[c, pl.ds(j * 16, 16)] = w_v[par, c, pl.ds(j * 16, 16)]
            return ()

        lax.fori_loop(0, CQ, snap, ())

        @pl.when(g + 2 < NIT)
        def _():
            fire_iw(g + 2, par)
        combine_write(g, par)

    lax.fori_loop(0, NIT, step, ())


def _sc_gather_combine(table, idx, wts):
    fn = pl.kernel(
        _sc_body,
        out_type=[
            jax.ShapeDtypeStruct((QH, 128), jnp.float32),
            jax.ShapeDtypeStruct((QH, 128), jnp.float32),
        ],
        mesh=plsc.VectorSubcoreMesh(core_axis_name="c", subcore_axis_name="s"),
        scratch_types=[
            pltpu.VMEM((2, CQ, K), jnp.int32),
            pltpu.VMEM((2, CQ, K), jnp.float32),
            pltpu.VMEM((CQ, K), jnp.float32),
            pltpu.VMEM((2, CQ * K, DH), jnp.float32),
            pltpu.VMEM((2, CQ, 128), jnp.float32),
            pltpu.SemaphoreType.DMA,
            pltpu.SemaphoreType.DMA,
            pltpu.SemaphoreType.DMA,
            pltpu.SemaphoreType.DMA,
        ],
        compiler_params=pltpu.CompilerParams(use_tc_tiling_on_sc=False),
    )
    return fn(table, idx, wts)


def _out_proj_body(xa_ref, xb_ref, wo_ref, bo_ref, o_ref):
    o_ref[...] = (
        jnp.dot(xa_ref[...], wo_ref[:128], preferred_element_type=jnp.float32,
                precision=DP)
        + jnp.dot(xb_ref[...], wo_ref[128:], preferred_element_type=jnp.float32,
                  precision=DP)
        + bo_ref[0]
    )


def _out_proj(xa, xb, w_out, b_out):
    return pl.pallas_call(
        _out_proj_body,
        grid=(QH // QBLK,),
        in_specs=[
            pl.BlockSpec((QBLK, 128), lambda i: (i, 0)),
            pl.BlockSpec((QBLK, 128), lambda i: (i, 0)),
            pl.BlockSpec((DM, DM), lambda i: (0, 0)),
            pl.BlockSpec((1, DM), lambda i: (0, 0)),
        ],
        out_specs=pl.BlockSpec((QBLK, DM), lambda i: (i, 0)),
        out_shape=jax.ShapeDtypeStruct((QH, DM), jnp.float32),
    )(xa, xb, w_out, b_out)


def kernel(hidden_states, encoder_hidden_states, reference_points,
           spatial_shapes, W_off, b_off, W_attn, b_attn, W_val, b_val,
           W_out, b_out):
    del spatial_shapes
    wo4 = W_off.reshape(DM, H, P, 2)
    w_big = jnp.concatenate(
        [wo4[..., 0].reshape(DM, H * P), wo4[..., 1].reshape(DM, H * P)],
        axis=1)
    bo4 = b_off.reshape(H, P, 2)
    b_big = jnp.concatenate(
        [bo4[..., 0].reshape(H * P), bo4[..., 1].reshape(H * P)]) - 0.5

    halves = []
    for b0 in range(B):
        table = _value_proj(encoder_hidden_states, W_val,
                            b_val.reshape(1, DM), b0)
        idx, wts = _sampling(hidden_states,
                             reference_points.reshape(B, Q, 2),
                             w_big, b_big.reshape(1, 2 * H * P),
                             W_attn, b_attn.reshape(1, H * P), b0)
        attn_a, attn_b = _sc_gather_combine(
            table.reshape((B // 4) * H * S * 4, DH),
            idx.reshape(QH, K), wts.reshape(QH, K))
        halves.append(_out_proj(attn_a, attn_b, W_out, b_out.reshape(1, DM)))
    return jnp.concatenate(halves, axis=0).reshape(B, Q, DM)
